# Initial kernel scaffold; baseline (speedup 1.0000x reference)
import functools

import jax
import jax.numpy as jnp
from jax import lax
from jax.experimental import pallas as pl
from jax.experimental.pallas import tpu as pltpu
from jax.experimental.pallas import tpu_sc as plsc

N_NODES = 50000
D_FEAT = 256
NUM_CLUSTERS = 12500

NSC = 2
NT = 16
LANES = 16
HD = D_FEAT // NSC
HV = HD // LANES
ROWS_PER_TILE = N_NODES // NT
RCHUNK = 125
NRCHUNK = ROWS_PER_TILE // RCHUNK
CPAD = 12544
CPT = CPAD // NT
OCH = 196
NOCH = CPT // OCH
OBASE_MAX = NUM_CLUSTERS - OCH


def _body(x_hbm, cm_hbm, out_hbm, acc, cacc, ids, xb, ones, sb, cb):
    c = lax.axis_index("c")
    s = lax.axis_index("s")

    zeros16 = jnp.zeros((LANES,), jnp.float32)
    ones16 = jnp.ones((LANES,), jnp.float32)

    def fill_ones(i, _):
        ones[i] = ones16
        return 0
    lax.fori_loop(0, RCHUNK, fill_ones, 0)

    def zero_sb(i, _):
        sb[i // HV, i % HV] = zeros16
        return 0
    lax.fori_loop(0, OCH * HV, zero_sb, 0)

    def zero_cb(i, _):
        cb[i] = zeros16
        return 0
    lax.fori_loop(0, OCH, zero_cb, 0)

    def zero_acc(k, _):
        base = s * CPT + k * OCH
        pltpu.sync_copy(sb, acc.at[pl.ds(base, OCH)])
        pltpu.sync_copy(cb, cacc.at[pl.ds(base, OCH)])
        return 0
    lax.fori_loop(0, NOCH, zero_acc, 0)

    pltpu.sync_copy(cm_hbm.at[pl.ds(s * NRCHUNK, NRCHUNK)], ids)

    plsc.subcore_barrier()

    def accum(j, _):
        off = s * ROWS_PER_TILE + j * RCHUNK
        pltpu.sync_copy(x_hbm.at[pl.ds(off, RCHUNK), c], xb)
        pltpu.sync_copy(xb, acc.at[ids.at[j]], add=True)
        pltpu.sync_copy(ones, cacc.at[ids.at[j]], add=True)
        return 0
    lax.fori_loop(0, NRCHUNK, accum, 0)

    plsc.subcore_barrier()

    def out_chunk(k, _):
        base = jnp.minimum(s * CPT + k * OCH, OBASE_MAX)
        pltpu.sync_copy(acc.at[pl.ds(base, OCH)], sb)
        pltpu.sync_copy(cacc.at[pl.ds(base, OCH)], cb)

        def div_row(r, _):
            inv = 1.0 / jnp.maximum(cb[r], 1.0)

            def div_vec(v, _):
                sb[r, v] = sb[r, v] * inv
                return 0
            lax.fori_loop(0, HV, div_vec, 0)
            return 0
        lax.fori_loop(0, OCH, div_row, 0)

        pltpu.sync_copy(sb, out_hbm.at[pl.ds(base, OCH), c])
        return 0
    lax.fori_loop(0, NOCH, out_chunk, 0)


@jax.jit
def _pooled(x4, cm2):
    mesh = plsc.VectorSubcoreMesh(core_axis_name="c", subcore_axis_name="s")
    f = functools.partial(
        pl.kernel,
        mesh=mesh,
        out_type=jax.ShapeDtypeStruct((NUM_CLUSTERS, NSC, HV, LANES),
                                      jnp.float32),
        scratch_types=[
            pltpu.VMEM_SHARED((CPAD, HV, LANES), jnp.float32),
            pltpu.VMEM_SHARED((CPAD, LANES), jnp.float32),
            pltpu.VMEM((NRCHUNK, RCHUNK), jnp.int32),
            pltpu.VMEM((RCHUNK, HV, LANES), jnp.float32),
            pltpu.VMEM((RCHUNK, LANES), jnp.float32),
            pltpu.VMEM((OCH, HV, LANES), jnp.float32),
            pltpu.VMEM((OCH, LANES), jnp.float32),
        ],
    )(_body)
    return f(x4, cm2)


def kernel(x, cluster_map, edge_index):
    x4 = x.reshape(N_NODES, NSC, HV, LANES)
    cm2 = cluster_map.reshape(N_NODES // RCHUNK, RCHUNK)
    out = _pooled(x4, cm2)
    return out.reshape(NUM_CLUSTERS, D_FEAT), edge_index


# speedup vs baseline: 2.1866x; 2.1866x over previous
import functools

import jax
import jax.numpy as jnp
from jax import lax
from jax.experimental import pallas as pl
from jax.experimental.pallas import tpu as pltpu
from jax.experimental.pallas import tpu_sc as plsc

N_NODES = 50000
D_FEAT = 256
NUM_CLUSTERS = 12500

NSC = 2
NT = 16
LANES = 16
HD = D_FEAT // NSC
HV = HD // LANES
ROWS_PER_TILE = N_NODES // NT
RCHUNK = 125
RPAD = 128
NRCHUNK = ROWS_PER_TILE // RCHUNK
NCHUNKS = N_NODES // RCHUNK
CW = 16
CPAD = 12544
CPT = CPAD // NT
ZCH = 112
NZCH = CPT // ZCH


def _sc_body(x_hbm, cm_hbm, sums_hbm, cnts_hbm, acc, cacc, ids, xb, ones):
    c = lax.axis_index("c")
    s = lax.axis_index("s")

    zeros16 = jnp.zeros((LANES,), jnp.float32)
    czero2 = jnp.zeros((2, CW), jnp.int16)
    cone2 = jnp.ones((2, CW), jnp.int16)

    def zero_xb(r, _):
        def zv(v, _):
            xb[2 * r, pl.ds(v * LANES, LANES)] = zeros16
            xb[2 * r + 1, pl.ds(v * LANES, LANES)] = zeros16
            return 0
        lax.fori_loop(0, HV, zv, 0)
        ones[pl.ds(2 * r, 2), :] = czero2
        return 0
    lax.fori_loop(0, RPAD // 2, zero_xb, 0)

    def zero_acc(k, _):
        base = s * CPT + k * ZCH
        pltpu.sync_copy(xb.at[pl.ds(0, ZCH)], acc.at[pl.ds(base, ZCH)])
        pltpu.sync_copy(ones.at[pl.ds(0, ZCH)], cacc.at[pl.ds(base, ZCH)])
        return 0
    lax.fori_loop(0, NZCH, zero_acc, 0)

    def fill_ones(i, _):
        ones[pl.ds(2 * i, 2), :] = cone2
        return 0
    lax.fori_loop(0, (RCHUNK + 1) // 2, fill_ones, 0)
    ones[pl.ds(RCHUNK, 2), :] = czero2

    plsc.subcore_barrier()

    def accum(j, _):
        chunk = s * NRCHUNK + j
        pltpu.sync_copy(cm_hbm.at[chunk], ids)
        pltpu.sync_copy(x_hbm.at[c, chunk], xb.at[pl.ds(0, RCHUNK)])
        pltpu.sync_copy(xb, acc.at[ids], add=True)
        pltpu.sync_copy(ones, cacc.at[ids], add=True)
        return 0
    lax.fori_loop(0, NRCHUNK, accum, 0)

    plsc.subcore_barrier()

    sl = pl.ds(s * CPT, CPT)
    pltpu.sync_copy(acc.at[sl], sums_hbm.at[c, sl])

    @pl.when(c == 0)
    def _():
        pltpu.sync_copy(cacc.at[sl], cnts_hbm.at[sl])


def _tc_divide(sums_ref, cnts_ref, out_ref):
    cnt = cnts_ref[:, 0:1].astype(jnp.int32) & 0xFFFF
    inv = 1.0 / jnp.maximum(cnt.astype(jnp.float32), 1.0)
    out_ref[:, :HD] = sums_ref[0] * inv
    out_ref[:, HD:] = sums_ref[1] * inv


@jax.jit
def _pooled(x4, cm3):
    mesh = plsc.VectorSubcoreMesh(core_axis_name="c", subcore_axis_name="s")
    f = functools.partial(
        pl.kernel,
        mesh=mesh,
        out_type=(
            jax.ShapeDtypeStruct((NSC, CPAD, HD), jnp.float32),
            jax.ShapeDtypeStruct((CPAD, CW), jnp.int16),
        ),
        scratch_types=[
            pltpu.VMEM_SHARED((CPAD, HD), jnp.float32),
            pltpu.VMEM_SHARED((CPAD, CW), jnp.int16),
            pltpu.VMEM((RPAD,), jnp.int32),
            pltpu.VMEM((RPAD, HD), jnp.float32),
            pltpu.VMEM((RPAD, CW), jnp.int16),
        ],
        compiler_params=pltpu.CompilerParams(
            use_tc_tiling_on_sc=False, needs_layout_passes=False
        ),
    )(_sc_body)
    sums, cnts = f(x4, cm3)

    out = pl.pallas_call(
        _tc_divide,
        grid=(NT,),
        in_specs=[
            pl.BlockSpec((NSC, CPT, HD), lambda i: (0, i, 0)),
            pl.BlockSpec((CPT, CW), lambda i: (i, 0)),
        ],
        out_specs=pl.BlockSpec((CPT, D_FEAT), lambda i: (i, 0)),
        out_shape=jax.ShapeDtypeStruct((CPAD, D_FEAT), jnp.float32),
    )(sums, cnts)
    return out


def kernel(x, cluster_map, edge_index):
    x4 = x.reshape(NCHUNKS, RCHUNK, NSC, HD).transpose(2, 0, 1, 3)
    cm2 = cluster_map.reshape(NCHUNKS, RCHUNK)
    cm3 = jnp.pad(cm2, ((0, 0), (0, RPAD - RCHUNK)), mode="edge")
    out = _pooled(x4, cm3)
    return out[:NUM_CLUSTERS], edge_index


# speedup vs baseline: 4.0568x; 1.8553x over previous
import functools

import jax
import jax.numpy as jnp
from jax import lax
from jax.experimental import pallas as pl
from jax.experimental.pallas import tpu as pltpu
from jax.experimental.pallas import tpu_sc as plsc

N_NODES = 50000
D_FEAT = 256
NUM_CLUSTERS = 12500

NSC = 2
NT = 16
LANES = 16
HD = D_FEAT // NSC
HV = HD // LANES
ROWS_PER_TILE = N_NODES // NT
RCHUNK = 125
RPAD = 128
NRCHUNK = ROWS_PER_TILE // RCHUNK
NCHUNKS = N_NODES // RCHUNK
SUB = 64
SUB1 = RCHUNK - SUB
CW = 16
CPAD = 12544
CPT = CPAD // NT
NZ = CPT // SUB
ZTAIL = CPT - NZ * SUB


def _sc_body(x_hbm, cm_hbm, sums_hbm, cnts_hbm,
             acc, cacc, ids, xba, xbb, ones_a, ones_b, sema, semb):
    c = lax.axis_index("c")
    s = lax.axis_index("s")
    cols = pl.ds(c * HD, HD)

    zeros16 = jnp.zeros((LANES,), jnp.float32)
    czero2 = jnp.zeros((2, CW), jnp.int16)
    cone2 = jnp.ones((2, CW), jnp.int16)

    def zero_bufs(r, _):
        def zv(v, _):
            sl = pl.ds(v * LANES, LANES)
            xba[r, sl] = zeros16
            xbb[r, sl] = zeros16
            return 0
        lax.fori_loop(0, HV, zv, 0)
        return 0
    lax.fori_loop(0, SUB, zero_bufs, 0)

    def zero_ones(i, _):
        ones_a[pl.ds(2 * i, 2), :] = czero2
        ones_b[pl.ds(2 * i, 2), :] = czero2
        return 0
    lax.fori_loop(0, SUB // 2, zero_ones, 0)

    def zero_acc(k, _):
        base = s * CPT + k * SUB
        pltpu.sync_copy(xba, acc.at[pl.ds(base, SUB)])
        pltpu.sync_copy(ones_a, cacc.at[pl.ds(base, SUB)])
        return 0
    lax.fori_loop(0, NZ, zero_acc, 0)
    tb = s * CPT + NZ * SUB
    pltpu.sync_copy(xba.at[pl.ds(0, ZTAIL)], acc.at[pl.ds(tb, ZTAIL)])
    pltpu.sync_copy(ones_a.at[pl.ds(0, ZTAIL)], cacc.at[pl.ds(tb, ZTAIL)])

    def fill_a(i, _):
        ones_a[pl.ds(2 * i, 2), :] = cone2
        return 0
    lax.fori_loop(0, SUB // 2, fill_a, 0)

    def fill_b(i, _):
        ones_b[pl.ds(2 * i, 2), :] = cone2
        return 0
    lax.fori_loop(0, (SUB1 + 1) // 2, fill_b, 0)
    ones_b[pl.ds(SUB1, 2), :] = czero2

    pltpu.sync_copy(cm_hbm.at[pl.ds(s * NRCHUNK, NRCHUNK)], ids)

    plsc.subcore_barrier()

    base_chunk = s * NRCHUNK
    pltpu.async_copy(
        x_hbm.at[base_chunk, pl.ds(0, SUB), cols], xba, sema
    )

    def accum(j, _):
        chunk = base_chunk + j
        ia = ids.at[j, pl.ds(0, SUB)]
        ib = ids.at[j, pl.ds(SUB, SUB)]
        pltpu.async_copy(
            x_hbm.at[chunk, pl.ds(SUB, SUB1), cols],
            xbb.at[pl.ds(0, SUB1)], semb,
        )
        pltpu.make_async_copy(
            x_hbm.at[chunk, pl.ds(0, SUB), cols], xba, sema
        ).wait()
        pltpu.sync_copy(xba, acc.at[ia], add=True)
        pltpu.sync_copy(ones_a, cacc.at[ia], add=True)
        nxt = jnp.minimum(chunk + 1, NCHUNKS - 1)
        pltpu.async_copy(x_hbm.at[nxt, pl.ds(0, SUB), cols], xba, sema)
        pltpu.make_async_copy(
            x_hbm.at[chunk, pl.ds(SUB, SUB1), cols],
            xbb.at[pl.ds(0, SUB1)], semb,
        ).wait()
        pltpu.sync_copy(xbb, acc.at[ib], add=True)
        pltpu.sync_copy(ones_b, cacc.at[ib], add=True)
        return 0
    lax.fori_loop(0, NRCHUNK, accum, 0)
    pltpu.make_async_copy(
        x_hbm.at[base_chunk, pl.ds(0, SUB), cols], xba, sema
    ).wait()

    plsc.subcore_barrier()

    sl = pl.ds(s * CPT, CPT)
    pltpu.sync_copy(acc.at[sl], sums_hbm.at[c, sl])

    @pl.when(c == 0)
    def _():
        pltpu.sync_copy(cacc.at[sl], cnts_hbm.at[sl])


def _tc_divide(sums_ref, cnts_ref, out_ref):
    cnt = cnts_ref[:, 0:1].astype(jnp.int32) & 0xFFFF
    inv = 1.0 / jnp.maximum(cnt.astype(jnp.float32), 1.0)
    out_ref[:, :HD] = sums_ref[0] * inv
    out_ref[:, HD:] = sums_ref[1] * inv


@jax.jit
def _pooled(x3, cm3):
    mesh = plsc.VectorSubcoreMesh(core_axis_name="c", subcore_axis_name="s")
    f = functools.partial(
        pl.kernel,
        mesh=mesh,
        out_type=(
            jax.ShapeDtypeStruct((NSC, CPAD, HD), jnp.float32),
            jax.ShapeDtypeStruct((CPAD, CW), jnp.int16),
        ),
        scratch_types=[
            pltpu.VMEM_SHARED((CPAD, HD), jnp.float32),
            pltpu.VMEM_SHARED((CPAD, CW), jnp.int16),
            pltpu.VMEM((NRCHUNK, RPAD), jnp.int32),
            pltpu.VMEM((SUB, HD), jnp.float32),
            pltpu.VMEM((SUB, HD), jnp.float32),
            pltpu.VMEM((SUB, CW), jnp.int16),
            pltpu.VMEM((SUB, CW), jnp.int16),
            pltpu.SemaphoreType.DMA,
            pltpu.SemaphoreType.DMA,
        ],
        compiler_params=pltpu.CompilerParams(
            use_tc_tiling_on_sc=False, needs_layout_passes=False
        ),
    )(_sc_body)
    sums, cnts = f(x3, cm3)

    out = pl.pallas_call(
        _tc_divide,
        grid=(NT,),
        in_specs=[
            pl.BlockSpec((NSC, CPT, HD), lambda i: (0, i, 0)),
            pl.BlockSpec((CPT, CW), lambda i: (i, 0)),
        ],
        out_specs=pl.BlockSpec((CPT, D_FEAT), lambda i: (i, 0)),
        out_shape=jax.ShapeDtypeStruct((CPAD, D_FEAT), jnp.float32),
    )(sums, cnts)
    return out


def kernel(x, cluster_map, edge_index):
    x3 = x.reshape(NCHUNKS, RCHUNK, D_FEAT)
    cm2 = cluster_map.reshape(NCHUNKS, RCHUNK)
    cm3 = jnp.pad(cm2, ((0, 0), (0, RPAD - RCHUNK)), mode="edge")
    out = _pooled(x3, cm3)
    return out[:NUM_CLUSTERS], edge_index


# speedup vs baseline: 4.3306x; 1.0675x over previous
import functools

import jax
import jax.numpy as jnp
from jax import lax
from jax.experimental import pallas as pl
from jax.experimental.pallas import tpu as pltpu
from jax.experimental.pallas import tpu_sc as plsc

N_NODES = 50000
D_FEAT = 256
NUM_CLUSTERS = 12500

NSC = 2
NT = 16
LANES = 16
HD = D_FEAT // NSC
HV = HD // LANES
ROWS_PER_TILE = N_NODES // NT
RCHUNK = 125
RPAD = 128
NRCHUNK = ROWS_PER_TILE // RCHUNK
NCHUNKS = N_NODES // RCHUNK
SUB = 64
SUB1 = RCHUNK - SUB
CW = 16
CPAD = 12544
CPT = CPAD // NT
NZ = CPT // SUB
ZTAIL = CPT - NZ * SUB


def _sc_body(x_hbm, cm_hbm, sums_hbm, cnts_hbm,
             acc, cacc, ids, xba, xbb, ones_a, ones_b, sema, semb):
    c = lax.axis_index("c")
    s = lax.axis_index("s")
    cols = pl.ds(c * HD, HD)

    zeros16 = jnp.zeros((LANES,), jnp.float32)
    czero2 = jnp.zeros((2, CW), jnp.int16)
    cone2 = jnp.ones((2, CW), jnp.int16)

    def zero_bufs(r, _):
        def zv(v, _):
            sl = pl.ds(v * LANES, LANES)
            xba[r, sl] = zeros16
            xbb[r, sl] = zeros16
            return 0
        lax.fori_loop(0, HV, zv, 0)
        return 0
    lax.fori_loop(0, SUB, zero_bufs, 0)

    def zero_ones(i, _):
        ones_a[pl.ds(2 * i, 2), :] = czero2
        ones_b[pl.ds(2 * i, 2), :] = czero2
        return 0
    lax.fori_loop(0, SUB // 2, zero_ones, 0)

    def zero_acc(k, _):
        base = s * CPT + k * SUB
        pltpu.sync_copy(xba, acc.at[pl.ds(base, SUB)])
        pltpu.sync_copy(ones_a, cacc.at[pl.ds(base, SUB)])
        return 0
    lax.fori_loop(0, NZ, zero_acc, 0)
    tb = s * CPT + NZ * SUB
    pltpu.sync_copy(xba.at[pl.ds(0, ZTAIL)], acc.at[pl.ds(tb, ZTAIL)])
    pltpu.sync_copy(ones_a.at[pl.ds(0, ZTAIL)], cacc.at[pl.ds(tb, ZTAIL)])

    def fill_a(i, _):
        ones_a[pl.ds(2 * i, 2), :] = cone2
        return 0
    lax.fori_loop(0, SUB // 2, fill_a, 0)

    def fill_b(i, _):
        ones_b[pl.ds(2 * i, 2), :] = cone2
        return 0
    lax.fori_loop(0, (SUB1 + 1) // 2, fill_b, 0)
    ones_b[pl.ds(SUB1, 2), :] = czero2

    pltpu.sync_copy(cm_hbm.at[pl.ds(s * NRCHUNK, NRCHUNK)], ids)

    plsc.subcore_barrier()

    base_chunk = s * NRCHUNK
    pltpu.async_copy(
        x_hbm.at[base_chunk, pl.ds(0, SUB), cols], xba, sema
    )

    def accum(j, _):
        chunk = base_chunk + j
        ia = ids.at[j, pl.ds(0, SUB)]
        ib = ids.at[j, pl.ds(SUB, SUB)]
        pltpu.async_copy(
            x_hbm.at[chunk, pl.ds(SUB, SUB1), cols],
            xbb.at[pl.ds(0, SUB1)], semb,
        )
        pltpu.make_async_copy(
            x_hbm.at[chunk, pl.ds(0, SUB), cols], xba, sema
        ).wait()
        pltpu.sync_copy(xba, acc.at[ia], add=True)
        pltpu.sync_copy(ones_a, cacc.at[ia], add=True)
        nxt = jnp.minimum(chunk + 1, NCHUNKS - 1)
        pltpu.async_copy(x_hbm.at[nxt, pl.ds(0, SUB), cols], xba, sema)
        pltpu.make_async_copy(
            x_hbm.at[chunk, pl.ds(SUB, SUB1), cols],
            xbb.at[pl.ds(0, SUB1)], semb,
        ).wait()
        pltpu.sync_copy(xbb, acc.at[ib], add=True)
        pltpu.sync_copy(ones_b, cacc.at[ib], add=True)
        return 0
    lax.fori_loop(0, NRCHUNK, accum, 0)
    pltpu.make_async_copy(
        x_hbm.at[base_chunk, pl.ds(0, SUB), cols], xba, sema
    ).wait()

    plsc.subcore_barrier()

    sl = pl.ds(s * CPT, CPT)
    pltpu.sync_copy(acc.at[sl], sums_hbm.at[c, sl])

    @pl.when(c == 0)
    def _():
        pltpu.sync_copy(cacc.at[sl], cnts_hbm.at[sl])


def _tc_divide(sums_ref, cnts_ref, out_ref):
    cnt = cnts_ref[:, 0:1].astype(jnp.int32) & 0xFFFF
    inv = 1.0 / jnp.maximum(cnt.astype(jnp.float32), 1.0)
    out_ref[:, :HD] = sums_ref[0] * inv
    out_ref[:, HD:] = sums_ref[1] * inv


@jax.jit
def _pooled(x3, cm3):
    mesh = plsc.VectorSubcoreMesh(core_axis_name="c", subcore_axis_name="s")
    f = functools.partial(
        pl.kernel,
        mesh=mesh,
        out_type=(
            jax.ShapeDtypeStruct((NSC, CPAD, HD), jnp.float32),
            jax.ShapeDtypeStruct((CPAD, CW), jnp.int16),
        ),
        scratch_types=[
            pltpu.VMEM_SHARED((CPAD, HD), jnp.float32),
            pltpu.VMEM_SHARED((CPAD, CW), jnp.int16),
            pltpu.VMEM((NRCHUNK, RPAD), jnp.int32),
            pltpu.VMEM((SUB, HD), jnp.float32),
            pltpu.VMEM((SUB, HD), jnp.float32),
            pltpu.VMEM((SUB, CW), jnp.int16),
            pltpu.VMEM((SUB, CW), jnp.int16),
            pltpu.SemaphoreType.DMA,
            pltpu.SemaphoreType.DMA,
        ],
        compiler_params=pltpu.CompilerParams(
            use_tc_tiling_on_sc=False, needs_layout_passes=False
        ),
    )(_sc_body)
    sums, cnts = f(x3, cm3)

    out = pl.pallas_call(
        _tc_divide,
        grid=(NT,),
        in_specs=[
            pl.BlockSpec((NSC, CPT, HD), lambda i: (0, i, 0)),
            pl.BlockSpec((CPT, CW), lambda i: (i, 0)),
        ],
        out_specs=pl.BlockSpec((CPT, D_FEAT), lambda i: (i, 0)),
        out_shape=jax.ShapeDtypeStruct((NUM_CLUSTERS, D_FEAT), jnp.float32),
    )(sums, cnts)
    return out


def kernel(x, cluster_map, edge_index):
    x3 = x.reshape(NCHUNKS, RCHUNK, D_FEAT)
    cm2 = cluster_map.reshape(NCHUNKS, RCHUNK)
    cm3 = jnp.pad(cm2, ((0, 0), (0, RPAD - RCHUNK)), mode="edge")
    return _pooled(x3, cm3), edge_index


# speedup vs baseline: 4.5312x; 1.0463x over previous
import functools

import jax
import jax.numpy as jnp
from jax import lax
from jax.experimental import pallas as pl
from jax.experimental.pallas import tpu as pltpu
from jax.experimental.pallas import tpu_sc as plsc

N_NODES = 50000
D_FEAT = 256
NUM_CLUSTERS = 12500

NSC = 2
NT = 16
LANES = 16
HD = D_FEAT // NSC
HV = HD // LANES
ROWS_PER_TILE = N_NODES // NT
RCHUNK = 125
RPAD = 128
NRCHUNK = ROWS_PER_TILE // RCHUNK
NCHUNKS = N_NODES // RCHUNK
SUB = 64
SUB1 = RCHUNK - SUB
CW = 16
CPAD = 12544
CPT = CPAD // NT
NZ = CPT // SUB
ZTAIL = CPT - NZ * SUB


def _sc_body(x_hbm, cm_hbm, sums_hbm, cnts_hbm,
             acc, cacc, ids, xba, xbb, ones, sema, semb, semc):
    c = lax.axis_index("c")
    s = lax.axis_index("s")
    cols = pl.ds(c * HD, HD)

    zeros16 = jnp.zeros((LANES,), jnp.float32)
    czero2 = jnp.zeros((2, CW), jnp.int16)
    cone2 = jnp.ones((2, CW), jnp.int16)

    def zero_bufs(r, _):
        def zv(v, _):
            sl = pl.ds(v * LANES, LANES)
            xba[r, sl] = zeros16
            xbb[r, sl] = zeros16
            return 0
        lax.fori_loop(0, HV, zv, 0)
        return 0
    lax.fori_loop(0, SUB, zero_bufs, 0)

    def zero_ones(i, _):
        ones[pl.ds(2 * i, 2), :] = czero2
        return 0
    lax.fori_loop(0, RPAD // 2, zero_ones, 0)

    def zero_acc(k, _):
        base = s * CPT + k * SUB
        pltpu.sync_copy(xba, acc.at[pl.ds(base, SUB)])
        pltpu.sync_copy(ones.at[pl.ds(0, SUB)], cacc.at[pl.ds(base, SUB)])
        return 0
    lax.fori_loop(0, NZ, zero_acc, 0)
    tb = s * CPT + NZ * SUB
    pltpu.sync_copy(xba.at[pl.ds(0, ZTAIL)], acc.at[pl.ds(tb, ZTAIL)])
    pltpu.sync_copy(ones.at[pl.ds(0, ZTAIL)], cacc.at[pl.ds(tb, ZTAIL)])

    def fill_ones(i, _):
        ones[pl.ds(2 * i, 2), :] = cone2
        return 0
    lax.fori_loop(0, (RCHUNK + 1) // 2, fill_ones, 0)
    ones[pl.ds(RCHUNK, 2), :] = czero2

    pltpu.sync_copy(cm_hbm.at[pl.ds(s * NRCHUNK, NRCHUNK)], ids)

    plsc.subcore_barrier()

    base_chunk = s * NRCHUNK
    pltpu.async_copy(
        x_hbm.at[base_chunk, pl.ds(0, SUB), cols], xba, sema
    )

    def accum(j, _):
        chunk = base_chunk + j
        ia = ids.at[j, pl.ds(0, SUB)]
        ib = ids.at[j, pl.ds(SUB, SUB)]
        pltpu.async_copy(
            x_hbm.at[chunk, pl.ds(SUB, SUB1), cols],
            xbb.at[pl.ds(0, SUB1)], semb,
        )
        pltpu.make_async_copy(
            x_hbm.at[chunk, pl.ds(0, SUB), cols], xba, sema
        ).wait()
        pltpu.sync_copy(xba, acc.at[ia], add=True)
        pltpu.async_copy(ones, cacc.at[ids.at[j]], semc, add=True)
        nxt = jnp.minimum(chunk + 1, NCHUNKS - 1)
        pltpu.async_copy(x_hbm.at[nxt, pl.ds(0, SUB), cols], xba, sema)
        pltpu.make_async_copy(
            x_hbm.at[chunk, pl.ds(SUB, SUB1), cols],
            xbb.at[pl.ds(0, SUB1)], semb,
        ).wait()
        pltpu.sync_copy(xbb, acc.at[ib], add=True)
        return 0
    lax.fori_loop(0, NRCHUNK, accum, 0)

    def drain_counts(j, _):
        pltpu.make_async_copy(ones, cacc.at[ids.at[j]], semc).wait()
        return 0
    lax.fori_loop(0, NRCHUNK, drain_counts, 0)
    pltpu.make_async_copy(
        x_hbm.at[base_chunk, pl.ds(0, SUB), cols], xba, sema
    ).wait()

    plsc.subcore_barrier()

    sl = pl.ds(s * CPT, CPT)
    pltpu.sync_copy(acc.at[sl], sums_hbm.at[c, sl])

    @pl.when(c == 0)
    def _():
        pltpu.sync_copy(cacc.at[sl], cnts_hbm.at[sl])


def _tc_divide(sums_ref, cnts_ref, out_ref):
    cnt = cnts_ref[:, 0:1].astype(jnp.int32) & 0xFFFF
    inv = 1.0 / jnp.maximum(cnt.astype(jnp.float32), 1.0)
    out_ref[:, :HD] = sums_ref[0] * inv
    out_ref[:, HD:] = sums_ref[1] * inv


@jax.jit
def _pooled(x3, cm3):
    mesh = plsc.VectorSubcoreMesh(core_axis_name="c", subcore_axis_name="s")
    f = functools.partial(
        pl.kernel,
        mesh=mesh,
        out_type=(
            jax.ShapeDtypeStruct((NSC, CPAD, HD), jnp.float32),
            jax.ShapeDtypeStruct((CPAD, CW), jnp.int16),
        ),
        scratch_types=[
            pltpu.VMEM_SHARED((CPAD, HD), jnp.float32),
            pltpu.VMEM_SHARED((CPAD, CW), jnp.int16),
            pltpu.VMEM((NRCHUNK, RPAD), jnp.int32),
            pltpu.VMEM((SUB, HD), jnp.float32),
            pltpu.VMEM((SUB, HD), jnp.float32),
            pltpu.VMEM((RPAD, CW), jnp.int16),
            pltpu.SemaphoreType.DMA,
            pltpu.SemaphoreType.DMA,
            pltpu.SemaphoreType.DMA,
        ],
        compiler_params=pltpu.CompilerParams(
            use_tc_tiling_on_sc=False, needs_layout_passes=False
        ),
    )(_sc_body)
    sums, cnts = f(x3, cm3)

    blk = 2 * CPT
    out = pl.pallas_call(
        _tc_divide,
        grid=(CPAD // blk,),
        in_specs=[
            pl.BlockSpec((NSC, blk, HD), lambda i: (0, i, 0)),
            pl.BlockSpec((blk, CW), lambda i: (i, 0)),
        ],
        out_specs=pl.BlockSpec((blk, D_FEAT), lambda i: (i, 0)),
        out_shape=jax.ShapeDtypeStruct((NUM_CLUSTERS, D_FEAT), jnp.float32),
    )(sums, cnts)
    return out


def kernel(x, cluster_map, edge_index):
    x3 = x.reshape(NCHUNKS, RCHUNK, D_FEAT)
    cm2 = cluster_map.reshape(NCHUNKS, RCHUNK)
    cm3 = jnp.pad(cm2, ((0, 0), (0, RPAD - RCHUNK)), mode="edge")
    return _pooled(x3, cm3), edge_index


# speedup vs baseline: 6.5658x; 1.4490x over previous
import functools

import jax
import jax.numpy as jnp
from jax import lax
from jax.experimental import pallas as pl
from jax.experimental.pallas import tpu as pltpu
from jax.experimental.pallas import tpu_sc as plsc

N_NODES = 50000
D_FEAT = 256
NUM_CLUSTERS = 12500

NSC = 2
NT = 16
LANES = 16
HD = D_FEAT // NSC
HV = HD // LANES
SL = 8
NRT = N_NODES // SL
RPT = 3128
RPT_LAST = N_NODES - (NT - 1) * RPT
SUB = 64
NSUB = 48
TAIL = RPT - NSUB * SUB
TAIL_L = RPT_LAST - NSUB * SUB
CW = 16
CPAD = 12544
CPT = CPAD // NT
NZ = CPT // SUB
ZTAIL = CPT - NZ * SUB


def _sc_body(x_hbm, cm_hbm, sums_hbm, cnts_hbm,
             acc, cacc, ids, xba, xbb, ones, sxa, sxb, semc):
    c = lax.axis_index("c")
    s = lax.axis_index("s")
    base = s * RPT
    base_rt = s * (RPT // SL)

    zeros16 = jnp.zeros((LANES,), jnp.float32)
    czero2 = jnp.zeros((2, CW), jnp.int16)
    cone2 = jnp.ones((2, CW), jnp.int16)

    def zero_bufs(r, _):
        def zv(v, _):
            xba[r, pl.ds(v * LANES, LANES)] = zeros16
            return 0
        lax.fori_loop(0, HV, zv, 0)
        ones[pl.ds(2 * r, 2), :] = czero2
        return 0
    lax.fori_loop(0, SUB, zero_bufs, 0)

    def zero_acc(k, _):
        b = s * CPT + k * SUB
        pltpu.sync_copy(xba, acc.at[pl.ds(b, SUB)])
        pltpu.sync_copy(ones, cacc.at[pl.ds(b, SUB)])
        return 0
    lax.fori_loop(0, NZ, zero_acc, 0)
    tb = s * CPT + NZ * SUB
    pltpu.sync_copy(xba.at[pl.ds(0, ZTAIL)], acc.at[pl.ds(tb, ZTAIL)])
    pltpu.sync_copy(ones.at[pl.ds(0, ZTAIL)], cacc.at[pl.ds(tb, ZTAIL)])

    def fill_ones(i, _):
        ones[pl.ds(2 * i, 2), :] = cone2
        return 0
    lax.fori_loop(0, SUB // 2, fill_ones, 0)

    @pl.when(s < NT - 1)
    def _():
        pltpu.sync_copy(cm_hbm.at[pl.ds(base, RPT)], ids)

    @pl.when(s == NT - 1)
    def _():
        pltpu.sync_copy(cm_hbm.at[pl.ds(base, RPT_LAST)],
                        ids.at[pl.ds(0, RPT_LAST)])

    plsc.subcore_barrier()

    def load_sub(k, buf, sem):
        def ld(m, _):
            pltpu.async_copy(x_hbm.at[base_rt + k * SL + m, c],
                             buf.at[pl.ds(SL * m, SL)], sem)
            return 0
        lax.fori_loop(0, SL, ld, 0)

    def wait_sub(buf, sem):
        def wt(m, _):
            pltpu.make_async_copy(x_hbm.at[base_rt, c],
                                  buf.at[pl.ds(SL * m, SL)], sem).wait()
            return 0
        lax.fori_loop(0, SL, wt, 0)

    load_sub(0, xba, sxa)

    def accum(j, _):
        ka = 2 * j
        kb = 2 * j + 1
        load_sub(kb, xbb, sxb)
        wait_sub(xba, sxa)
        pltpu.sync_copy(xba, acc.at[ids.at[pl.ds(ka * SUB, SUB)]], add=True)
        pltpu.async_copy(ones, cacc.at[ids.at[pl.ds(ka * SUB, SUB)]],
                         semc, add=True)
        kn = jnp.minimum(ka + 2, NSUB - 1)
        load_sub(kn, xba, sxa)
        wait_sub(xbb, sxb)
        pltpu.sync_copy(xbb, acc.at[ids.at[pl.ds(kb * SUB, SUB)]], add=True)
        pltpu.async_copy(ones, cacc.at[ids.at[pl.ds(kb * SUB, SUB)]],
                         semc, add=True)
        return 0
    lax.fori_loop(0, NSUB // 2, accum, 0)
    wait_sub(xba, sxa)

    def drain_counts(k, _):
        pltpu.make_async_copy(ones, cacc.at[ids.at[pl.ds(0, SUB)]],
                              semc).wait()
        return 0
    lax.fori_loop(0, NSUB, drain_counts, 0)

    trt = base_rt + NSUB * SL

    @pl.when(s < NT - 1)
    def _():
        def ld(m, _):
            pltpu.async_copy(x_hbm.at[trt + m, c],
                             xba.at[pl.ds(SL * m, SL)], sxa)
            return 0
        lax.fori_loop(0, TAIL // SL, ld, 0)

        def wt(m, _):
            pltpu.make_async_copy(x_hbm.at[trt, c],
                                  xba.at[pl.ds(SL * m, SL)], sxa).wait()
            return 0
        lax.fori_loop(0, TAIL // SL, wt, 0)
        isl = ids.at[pl.ds(NSUB * SUB, TAIL)]
        pltpu.sync_copy(xba.at[pl.ds(0, TAIL)], acc.at[isl], add=True)
        pltpu.sync_copy(ones.at[pl.ds(0, TAIL)], cacc.at[isl], add=True)

    @pl.when(s == NT - 1)
    def _():
        pltpu.sync_copy(x_hbm.at[trt, c], xba.at[pl.ds(0, TAIL_L)])
        isl = ids.at[pl.ds(NSUB * SUB, TAIL_L)]
        pltpu.sync_copy(xba.at[pl.ds(0, TAIL_L)], acc.at[isl], add=True)
        pltpu.sync_copy(ones.at[pl.ds(0, TAIL_L)], cacc.at[isl], add=True)

    plsc.subcore_barrier()

    sl = pl.ds(s * CPT, CPT)
    pltpu.sync_copy(acc.at[sl], sums_hbm.at[c, sl])

    @pl.when(c == 0)
    def _():
        pltpu.sync_copy(cacc.at[sl], cnts_hbm.at[sl])


def _tc_divide(sums_ref, cnts_ref, out_ref):
    cnt = cnts_ref[:, 0:1].astype(jnp.int32) & 0xFFFF
    inv = 1.0 / jnp.maximum(cnt.astype(jnp.float32), 1.0)
    out_ref[:, :HD] = sums_ref[0] * inv
    out_ref[:, HD:] = sums_ref[1] * inv


@jax.jit
def _pooled(x4, cm):
    mesh = plsc.VectorSubcoreMesh(core_axis_name="c", subcore_axis_name="s")
    f = functools.partial(
        pl.kernel,
        mesh=mesh,
        out_type=(
            jax.ShapeDtypeStruct((NSC, CPAD, HD), jnp.float32),
            jax.ShapeDtypeStruct((CPAD, CW), jnp.int16),
        ),
        scratch_types=[
            pltpu.VMEM_SHARED((CPAD, HD), jnp.float32),
            pltpu.VMEM_SHARED((CPAD, CW), jnp.int16),
            pltpu.VMEM((RPT,), jnp.int32),
            pltpu.VMEM((SUB, HD), jnp.float32),
            pltpu.VMEM((SUB, HD), jnp.float32),
            pltpu.VMEM((SUB, CW), jnp.int16),
            pltpu.SemaphoreType.DMA,
            pltpu.SemaphoreType.DMA,
            pltpu.SemaphoreType.DMA,
        ],
        compiler_params=pltpu.CompilerParams(
            use_tc_tiling_on_sc=False, needs_layout_passes=False
        ),
    )(_sc_body)
    sums, cnts = f(x4, cm)

    blk = 2 * CPT
    out = pl.pallas_call(
        _tc_divide,
        grid=(CPAD // blk,),
        in_specs=[
            pl.BlockSpec((NSC, blk, HD), lambda i: (0, i, 0)),
            pl.BlockSpec((blk, CW), lambda i: (i, 0)),
        ],
        out_specs=pl.BlockSpec((blk, D_FEAT), lambda i: (i, 0)),
        out_shape=jax.ShapeDtypeStruct((NUM_CLUSTERS, D_FEAT), jnp.float32),
    )(sums, cnts)
    return out


def kernel(x, cluster_map, edge_index):
    x4 = x.reshape(NRT, SL, NSC, HD).transpose(0, 2, 1, 3)
    return _pooled(x4, cluster_map), edge_index
